# baseline (speedup 1.0000x reference)
import functools

import jax
import jax.numpy as jnp
from jax import lax
from jax.experimental import pallas as pl
from jax.experimental.pallas import tpu as pltpu
from jax.experimental.pallas import tpu_sc as plsc

VOCAB = 1000000
D_MODEL = 64
BATCH = 4096
SEQ = 200

CHUNK = 128
N_TOKENS = BATCH * SEQ
_info = plsc.get_sparse_core_info()
NC, NS = _info.num_cores, _info.num_subcores
NW = NC * NS
CHUNKS_PER_W = N_TOKENS // (NW * CHUNK)
ROWS_PER_W = CHUNKS_PER_W * CHUNK


def _gather_kernel(w_hbm, idx_hbm, out_hbm, idx_v, rows_v, sem):
    wid = lax.axis_index("s") * NC + lax.axis_index("c")
    pltpu.sync_copy(idx_hbm.at[pl.ds(wid * CHUNKS_PER_W, CHUNKS_PER_W)], idx_v)
    base = wid * ROWS_PER_W

    def step(j, carry):
        pltpu.async_copy(w_hbm.at[idx_v.at[j]], rows_v, sem).wait()
        pltpu.sync_copy(rows_v, out_hbm.at[pl.ds(base + j * CHUNK, CHUNK)])
        return carry

    lax.fori_loop(0, CHUNKS_PER_W, step, 0)


@jax.jit
def _embed(token_ids, weight):
    idx2d = token_ids.reshape(NW * CHUNKS_PER_W, CHUNK)
    mesh = plsc.VectorSubcoreMesh(core_axis_name="c", subcore_axis_name="s")
    out = pl.kernel(
        _gather_kernel,
        mesh=mesh,
        out_type=jax.ShapeDtypeStruct((N_TOKENS, D_MODEL), jnp.float32),
        scratch_types=[
            pltpu.VMEM((CHUNKS_PER_W, CHUNK), jnp.int32),
            pltpu.VMEM((CHUNK, D_MODEL), jnp.float32),
            pltpu.SemaphoreType.DMA,
        ],
        compiler_params=pltpu.CompilerParams(use_tc_tiling_on_sc=False),
    )(weight, idx2d)
    return out.reshape(BATCH, SEQ, D_MODEL)


def kernel(token_ids, weight):
    return _embed(token_ids, weight)


# speedup vs baseline: 1.1149x; 1.1149x over previous
import jax
import jax.numpy as jnp
from jax import lax
from jax.experimental import pallas as pl
from jax.experimental.pallas import tpu as pltpu
from jax.experimental.pallas import tpu_sc as plsc

VOCAB = 1000000
D_MODEL = 64
BATCH = 4096
SEQ = 200

CHUNK = 128
NBUF = 8
N_TOKENS = BATCH * SEQ
_info = plsc.get_sparse_core_info()
NC, NS = _info.num_cores, _info.num_subcores
NW = NC * NS
CHUNKS_PER_W = N_TOKENS // (NW * CHUNK)
ROWS_PER_W = CHUNKS_PER_W * CHUNK
NGROUPS = CHUNKS_PER_W // NBUF


def _gather_kernel(w_hbm, idx_hbm, out_hbm, idx_v, rows_v, gsem, ssem):
    wid = lax.axis_index("s") * NC + lax.axis_index("c")
    pltpu.sync_copy(idx_hbm.at[pl.ds(wid * CHUNKS_PER_W, CHUNKS_PER_W)], idx_v)
    base = wid * ROWS_PER_W

    def start_gather(j, b):
        pltpu.async_copy(w_hbm.at[idx_v.at[j]], rows_v.at[b], gsem.at[b])

    def wait_gather(b):
        pltpu.make_async_copy(
            w_hbm.at[idx_v.at[0]], rows_v.at[b], gsem.at[b]).wait()

    def start_store(j, b):
        pltpu.async_copy(
            rows_v.at[b], out_hbm.at[pl.ds(base + j * CHUNK, CHUNK)],
            ssem.at[b])

    def wait_store(b):
        pltpu.make_async_copy(
            rows_v.at[b], out_hbm.at[pl.ds(base, CHUNK)], ssem.at[b]).wait()

    for b in range(NBUF):
        start_gather(b, b)

    def group(g, carry):
        for b in range(NBUF):
            wait_gather(b)
            start_store(g * NBUF + b, b)
        for b in range(NBUF):
            wait_store(b)
            start_gather((g + 1) * NBUF + b, b)
        return carry

    lax.fori_loop(0, NGROUPS - 1, group, 0)

    for b in range(NBUF):
        wait_gather(b)
        start_store((NGROUPS - 1) * NBUF + b, b)
    for b in range(NBUF):
        wait_store(b)


@jax.jit
def _embed(token_ids, weight):
    idx2d = token_ids.reshape(NW * CHUNKS_PER_W, CHUNK)
    mesh = plsc.VectorSubcoreMesh(core_axis_name="c", subcore_axis_name="s")
    out = pl.kernel(
        _gather_kernel,
        mesh=mesh,
        out_type=jax.ShapeDtypeStruct((N_TOKENS, D_MODEL), jnp.float32),
        scratch_types=[
            pltpu.VMEM((CHUNKS_PER_W, CHUNK), jnp.int32),
            pltpu.VMEM((NBUF, CHUNK, D_MODEL), jnp.float32),
            pltpu.SemaphoreType.DMA((NBUF,)),
            pltpu.SemaphoreType.DMA((NBUF,)),
        ],
        compiler_params=pltpu.CompilerParams(use_tc_tiling_on_sc=False),
    )(weight, idx2d)
    return out.reshape(BATCH, SEQ, D_MODEL)


def kernel(token_ids, weight):
    return _embed(token_ids, weight)
